# baseline (speedup 1.0000x reference)
import functools

import jax
import jax.numpy as jnp
from jax import lax
from jax.experimental import pallas as pl
from jax.experimental.pallas import tpu as pltpu
from jax.experimental.pallas import tpu_sc as plsc

N_EDGE = 160000
D = 256
NC = 2
NS = 16
NW = NC * NS
PER_W = N_EDGE // NW
C = 128
N_FULL = PER_W // C
TAIL_OFF = PER_W - C
LANES = 16


def _sc_body(z_hbm, src_hbm, dst_hbm, out_hbm,
             idx_s, idx_d, rows_s, rows_d, out_v, sem_s, sem_d):
    wid = lax.axis_index("s") * NC + lax.axis_index("c")
    wbase = wid * PER_W

    def chunk_body(i, carry):
        base = wbase + jnp.where(i < N_FULL, i * C, TAIL_OFF)
        pltpu.sync_copy(src_hbm.at[pl.ds(base, C)], idx_s)
        pltpu.sync_copy(dst_hbm.at[pl.ds(base, C)], idx_d)
        cp_s = pltpu.async_copy(z_hbm.at[idx_s], rows_s, sem_s)
        cp_d = pltpu.async_copy(z_hbm.at[idx_d], rows_d, sem_d)
        cp_s.wait()
        cp_d.wait()

        def group_body(g, carry2):
            lanes = g * LANES + lax.iota(jnp.int32, LANES)

            def feat_body(kb, acc):
                for j in range(8):
                    kv = jnp.full((LANES,), kb * 8 + j, jnp.int32)
                    vs = plsc.load_gather(rows_s, [lanes, kv])
                    vd = plsc.load_gather(rows_d, [lanes, kv])
                    acc = acc + vs * vd
                return acc

            acc = lax.fori_loop(0, D // 8, feat_body,
                                jnp.zeros((LANES,), jnp.float32))
            out_v[pl.ds(g * LANES, LANES)] = acc
            return carry2

        lax.fori_loop(0, C // LANES, group_body, 0)
        pltpu.sync_copy(out_v, out_hbm.at[pl.ds(base, C)])
        return carry

    lax.fori_loop(0, N_FULL + 1, chunk_body, 0)


@functools.partial(jax.jit, donate_argnums=())
def _link_predict(z, src, dst):
    mesh = plsc.VectorSubcoreMesh(core_axis_name="c", subcore_axis_name="s")
    run = pl.kernel(
        _sc_body,
        out_type=jax.ShapeDtypeStruct((N_EDGE,), jnp.float32),
        mesh=mesh,
        scratch_types=[
            pltpu.VMEM((C,), jnp.int32),
            pltpu.VMEM((C,), jnp.int32),
            pltpu.VMEM((C, D), jnp.float32),
            pltpu.VMEM((C, D), jnp.float32),
            pltpu.VMEM((C,), jnp.float32),
            pltpu.SemaphoreType.DMA,
            pltpu.SemaphoreType.DMA,
        ],
        compiler_params=pltpu.CompilerParams(
            use_tc_tiling_on_sc=False, needs_layout_passes=False),
    )
    return run(z, src, dst)


def kernel(z, edge):
    src = edge[:, 0].astype(jnp.int32)
    dst = edge[:, 1].astype(jnp.int32)
    return _link_predict(z, src, dst)


# speedup vs baseline: 1.0817x; 1.0817x over previous
import functools

import jax
import jax.numpy as jnp
from jax import lax
from jax.experimental import pallas as pl
from jax.experimental.pallas import tpu as pltpu
from jax.experimental.pallas import tpu_sc as plsc

N_EDGE = 160000
D = 256
NC = 2
NS = 16
NW = NC * NS
PER_W = N_EDGE // NW
C = 64
N_FULL = PER_W // C
TAIL_OFF = PER_W - C
NCH = N_FULL + 2
LANES = 16


def _sc_body(z_hbm, src_hbm, dst_hbm, out_hbm,
             idx_s0, idx_d0, idx_s1, idx_d1,
             rows_s0, rows_d0, rows_s1, rows_d1,
             out_v, sem_s0, sem_d0, sem_s1, sem_d1):
    wid = lax.axis_index("s") * NC + lax.axis_index("c")
    wbase = wid * PER_W
    idx_s = (idx_s0, idx_s1)
    idx_d = (idx_d0, idx_d1)
    rows_s = (rows_s0, rows_s1)
    rows_d = (rows_d0, rows_d1)
    sem_s = (sem_s0, sem_s1)
    sem_d = (sem_d0, sem_d1)

    def chunk_base(c):
        return wbase + jnp.where(c < N_FULL, c * C, TAIL_OFF)

    def issue(c, b):
        base = chunk_base(c)
        pltpu.sync_copy(src_hbm.at[pl.ds(base, C)], idx_s[b])
        pltpu.sync_copy(dst_hbm.at[pl.ds(base, C)], idx_d[b])
        pltpu.async_copy(z_hbm.at[idx_s[b]], rows_s[b], sem_s[b])
        pltpu.async_copy(z_hbm.at[idx_d[b]], rows_d[b], sem_d[b])

    for b in range(2):
        issue(b, b)

    def pair_body(i, carry):
        for b in range(2):
            c = i * 2 + b
            pltpu.make_async_copy(z_hbm.at[idx_s[b]], rows_s[b],
                                  sem_s[b]).wait()
            pltpu.make_async_copy(z_hbm.at[idx_d[b]], rows_d[b],
                                  sem_d[b]).wait()
            rs, rd = rows_s[b], rows_d[b]
            for g in range(C // LANES):
                lanes = g * LANES + lax.iota(jnp.int32, LANES)

                def feat_body(kb, accs, _lanes=lanes, _rs=rs, _rd=rd):
                    a0, a1, a2, a3 = accs
                    prods = []
                    for j in range(16):
                        kv = jnp.full((LANES,), kb * 16 + j, jnp.int32)
                        vs = plsc.load_gather(_rs, [_lanes, kv])
                        vd = plsc.load_gather(_rd, [_lanes, kv])
                        prods.append(vs * vd)
                    a0 = a0 + ((prods[0] + prods[4]) + (prods[8] + prods[12]))
                    a1 = a1 + ((prods[1] + prods[5]) + (prods[9] + prods[13]))
                    a2 = a2 + ((prods[2] + prods[6]) + (prods[10] + prods[14]))
                    a3 = a3 + ((prods[3] + prods[7]) + (prods[11] + prods[15]))
                    return (a0, a1, a2, a3)

                zero = jnp.zeros((LANES,), jnp.float32)
                a0, a1, a2, a3 = lax.fori_loop(
                    0, D // 16, feat_body, (zero, zero, zero, zero))
                out_v[pl.ds(g * LANES, LANES)] = (a0 + a1) + (a2 + a3)
            base = chunk_base(c)
            pltpu.sync_copy(out_v, out_hbm.at[pl.ds(base, C)])

            @pl.when(c + 2 < NCH)
            def _():
                issue(c + 2, b)

        return carry

    lax.fori_loop(0, NCH // 2, pair_body, 0)


@jax.jit
def _link_predict(z, src, dst):
    mesh = plsc.VectorSubcoreMesh(core_axis_name="c", subcore_axis_name="s")
    run = pl.kernel(
        _sc_body,
        out_type=jax.ShapeDtypeStruct((N_EDGE,), jnp.float32),
        mesh=mesh,
        scratch_types=[
            pltpu.VMEM((C,), jnp.int32),
            pltpu.VMEM((C,), jnp.int32),
            pltpu.VMEM((C,), jnp.int32),
            pltpu.VMEM((C,), jnp.int32),
            pltpu.VMEM((C, D), jnp.float32),
            pltpu.VMEM((C, D), jnp.float32),
            pltpu.VMEM((C, D), jnp.float32),
            pltpu.VMEM((C, D), jnp.float32),
            pltpu.VMEM((C,), jnp.float32),
            pltpu.SemaphoreType.DMA,
            pltpu.SemaphoreType.DMA,
            pltpu.SemaphoreType.DMA,
            pltpu.SemaphoreType.DMA,
        ],
        compiler_params=pltpu.CompilerParams(
            use_tc_tiling_on_sc=False, needs_layout_passes=False),
    )
    return run(z, src, dst)


def kernel(z, edge):
    src = edge[:, 0].astype(jnp.int32)
    dst = edge[:, 1].astype(jnp.int32)
    return _link_predict(z, src, dst)


# speedup vs baseline: 6.8094x; 6.2953x over previous
import functools

import jax
import jax.numpy as jnp
from jax import lax
from jax.experimental import pallas as pl
from jax.experimental.pallas import tpu as pltpu
from jax.experimental.pallas import tpu_sc as plsc

N_EDGE = 160000
D = 256
NC = 2
NS = 16
NW = NC * NS
PER_W = N_EDGE // NW
C = 64
N_FULL = PER_W // C
TAIL_OFF = PER_W - C
NCH = N_FULL + 2
LANES = 16


def _sc_body(z_hbm, src_hbm, dst_hbm, out_hbm,
             idx_s0, idx_d0, idx_s1, idx_d1,
             rows_s0, rows_d0, rows_s1, rows_d1,
             out_v, sem_s0, sem_d0, sem_s1, sem_d1):
    wid = lax.axis_index("s") * NC + lax.axis_index("c")
    wbase = wid * PER_W
    idx_s = (idx_s0, idx_s1)
    idx_d = (idx_d0, idx_d1)
    rows_s = (rows_s0, rows_s1)
    rows_d = (rows_d0, rows_d1)
    sem_s = (sem_s0, sem_s1)
    sem_d = (sem_d0, sem_d1)

    def chunk_base(c):
        return wbase + jnp.where(c < N_FULL, c * C, TAIL_OFF)

    def issue(c, b):
        base = chunk_base(c)
        pltpu.sync_copy(src_hbm.at[pl.ds(base, C)], idx_s[b])
        pltpu.sync_copy(dst_hbm.at[pl.ds(base, C)], idx_d[b])
        pltpu.async_copy(z_hbm.at[idx_s[b]], rows_s[b], sem_s[b])
        pltpu.async_copy(z_hbm.at[idx_d[b]], rows_d[b], sem_d[b])

    for b in range(2):
        issue(b, b)

    def pair_body(i, carry):
        for b in range(2):
            c = i * 2 + b
            pltpu.make_async_copy(z_hbm.at[idx_s[b]], rows_s[b],
                                  sem_s[b]).wait()
            pltpu.make_async_copy(z_hbm.at[idx_d[b]], rows_d[b],
                                  sem_d[b]).wait()
            rs, rd = rows_s[b], rows_d[b]
            for g in range(C // LANES):
                lanes = g * LANES + lax.iota(jnp.int32, LANES)

                def feat_body(kb, carry, _lanes=lanes, _rs=rs, _rd=rd):
                    kv, a0, a1, a2, a3 = carry
                    accs = [a0, a1, a2, a3]
                    for j in range(16):
                        vs = plsc.load_gather(_rs, [_lanes, kv])
                        vd = plsc.load_gather(_rd, [_lanes, kv])
                        accs[j % 4] = accs[j % 4] + vs * vd
                        kv = (kv + 1) & (D - 1)
                    return (kv, *accs)

                zero = jnp.zeros((LANES,), jnp.float32)
                kv0 = lax.iota(jnp.int32, LANES)
                _, a0, a1, a2, a3 = lax.fori_loop(
                    0, D // 16, feat_body, (kv0, zero, zero, zero, zero))
                out_v[pl.ds(g * LANES, LANES)] = (a0 + a1) + (a2 + a3)
            base = chunk_base(c)
            pltpu.sync_copy(out_v, out_hbm.at[pl.ds(base, C)])

            @pl.when(c + 2 < NCH)
            def _():
                issue(c + 2, b)

        return carry

    lax.fori_loop(0, NCH // 2, pair_body, 0)


@jax.jit
def _link_predict(z, src, dst):
    mesh = plsc.VectorSubcoreMesh(core_axis_name="c", subcore_axis_name="s")
    run = pl.kernel(
        _sc_body,
        out_type=jax.ShapeDtypeStruct((N_EDGE,), jnp.float32),
        mesh=mesh,
        scratch_types=[
            pltpu.VMEM((C,), jnp.int32),
            pltpu.VMEM((C,), jnp.int32),
            pltpu.VMEM((C,), jnp.int32),
            pltpu.VMEM((C,), jnp.int32),
            pltpu.VMEM((C, D), jnp.float32),
            pltpu.VMEM((C, D), jnp.float32),
            pltpu.VMEM((C, D), jnp.float32),
            pltpu.VMEM((C, D), jnp.float32),
            pltpu.VMEM((C,), jnp.float32),
            pltpu.SemaphoreType.DMA,
            pltpu.SemaphoreType.DMA,
            pltpu.SemaphoreType.DMA,
            pltpu.SemaphoreType.DMA,
        ],
        compiler_params=pltpu.CompilerParams(
            use_tc_tiling_on_sc=False, needs_layout_passes=False),
    )
    return run(z, src, dst)


def kernel(z, edge):
    src = edge[:, 0].astype(jnp.int32)
    dst = edge[:, 1].astype(jnp.int32)
    return _link_predict(z, src, dst)


# speedup vs baseline: 8.5781x; 1.2597x over previous
import jax
import jax.numpy as jnp
from jax import lax
from jax.experimental import pallas as pl
from jax.experimental.pallas import tpu as pltpu
from jax.experimental.pallas import tpu_sc as plsc

N_EDGE = 160000
D = 256
NC = 2
NS = 16
NW = NC * NS
PER_W = N_EDGE // NW
C = 96
N_FULL = PER_W // C
TAIL_OFF = PER_W - C
NCH = N_FULL + 2
LANES = 16


def _sc_body(z_hbm, src_hbm, dst_hbm, out_hbm,
             srcv, dstv, out_full,
             rows_s0, rows_d0, rows_s1, rows_d1,
             sem_s0, sem_d0, sem_s1, sem_d1):
    wid = lax.axis_index("s") * NC + lax.axis_index("c")
    wbase = wid * PER_W
    rows_s = (rows_s0, rows_s1)
    rows_d = (rows_d0, rows_d1)
    sem_s = (sem_s0, sem_s1)
    sem_d = (sem_d0, sem_d1)

    pltpu.sync_copy(src_hbm.at[pl.ds(wbase, PER_W)], srcv)
    pltpu.sync_copy(dst_hbm.at[pl.ds(wbase, PER_W)], dstv)

    def chunk_off(c):
        return jnp.where(c < N_FULL, c * C, TAIL_OFF)

    def issue(c, b):
        off = chunk_off(c)
        pltpu.async_copy(z_hbm.at[srcv.at[pl.ds(off, C)]], rows_s[b],
                         sem_s[b])
        pltpu.async_copy(z_hbm.at[dstv.at[pl.ds(off, C)]], rows_d[b],
                         sem_d[b])

    def drain(c, b):
        off = chunk_off(c)
        pltpu.make_async_copy(z_hbm.at[srcv.at[pl.ds(off, C)]], rows_s[b],
                              sem_s[b]).wait()
        pltpu.make_async_copy(z_hbm.at[dstv.at[pl.ds(off, C)]], rows_d[b],
                              sem_d[b]).wait()

    for b in range(2):
        issue(b, b)

    def pair_body(i, carry):
        for b in range(2):
            c = i * 2 + b
            drain(c, b)
            rs, rd = rows_s[b], rows_d[b]
            off = chunk_off(c)

            def group_body(g, carry2, _rs=rs, _rd=rd, _off=off):
                lanes = g * LANES + lax.iota(jnp.int32, LANES)

                def feat_body(kb, carry3):
                    kv, a0, a1, a2, a3 = carry3
                    accs = [a0, a1, a2, a3]
                    for j in range(16):
                        vs = plsc.load_gather(_rs, [lanes, kv])
                        vd = plsc.load_gather(_rd, [lanes, kv])
                        accs[j % 4] = accs[j % 4] + vs * vd
                        kv = (kv + 1) & (D - 1)
                    return (kv, *accs)

                zero = jnp.zeros((LANES,), jnp.float32)
                kv0 = lax.iota(jnp.int32, LANES)
                _, a0, a1, a2, a3 = lax.fori_loop(
                    0, D // 16, feat_body, (kv0, zero, zero, zero, zero))
                out_full[pl.ds(_off + g * LANES, LANES)] = (a0 + a1) + (a2 + a3)
                return carry2

            lax.fori_loop(0, C // LANES, group_body, 0)

            @pl.when(c + 2 < NCH)
            def _():
                issue(c + 2, b)

        return carry

    lax.fori_loop(0, NCH // 2, pair_body, 0)
    pltpu.sync_copy(out_full, out_hbm.at[pl.ds(wbase, PER_W)])


@jax.jit
def _link_predict(z, src, dst):
    mesh = plsc.VectorSubcoreMesh(core_axis_name="c", subcore_axis_name="s")
    run = pl.kernel(
        _sc_body,
        out_type=jax.ShapeDtypeStruct((N_EDGE,), jnp.float32),
        mesh=mesh,
        scratch_types=[
            pltpu.VMEM((PER_W,), jnp.int32),
            pltpu.VMEM((PER_W,), jnp.int32),
            pltpu.VMEM((PER_W,), jnp.float32),
            pltpu.VMEM((C, D), jnp.float32),
            pltpu.VMEM((C, D), jnp.float32),
            pltpu.VMEM((C, D), jnp.float32),
            pltpu.VMEM((C, D), jnp.float32),
            pltpu.SemaphoreType.DMA,
            pltpu.SemaphoreType.DMA,
            pltpu.SemaphoreType.DMA,
            pltpu.SemaphoreType.DMA,
        ],
        compiler_params=pltpu.CompilerParams(
            use_tc_tiling_on_sc=False, needs_layout_passes=False),
    )
    return run(z, src, dst)


def kernel(z, edge):
    src = edge[:, 0].astype(jnp.int32)
    dst = edge[:, 1].astype(jnp.int32)
    return _link_predict(z, src, dst)
